# Initial kernel scaffold; baseline (speedup 1.0000x reference)
import functools

import jax
import jax.numpy as jnp
from jax import lax
from jax.experimental import pallas as pl
from jax.experimental.pallas import tpu as pltpu
from jax.experimental.pallas import tpu_sc as plsc

F32 = jnp.float32
I32 = jnp.int32

K = 128
LANES = 16

_MESH = plsc.VectorSubcoreMesh(core_axis_name="c", subcore_axis_name="s")


def _build_sc_count(NP, ACC_R, CHC):
    @functools.partial(
        pl.kernel,
        mesh=_MESH,
        out_type=jax.ShapeDtypeStruct((2, NP, LANES), F32),
        scratch_types=[
            pltpu.VMEM_SHARED((ACC_R, LANES), F32),
            pltpu.VMEM((CHC, K), I32),
            pltpu.VMEM((K, LANES), F32),
            pltpu.VMEM((LANES, LANES), F32),
        ],
    )
    def sc_count(dst_hbm, out_hbm, acc, dst_v, ones_v, zbuf):
        c = lax.axis_index("c")
        s = lax.axis_index("s")
        w = c * 16 + s
        pltpu.sync_copy(dst_hbm.at[w], dst_v)
        ov = jnp.ones((LANES,), F32)
        zv = jnp.zeros((LANES,), F32)
        for r in range(K):
            ones_v[r, pl.ds(0, LANES)] = ov
        for r in range(LANES):
            zbuf[r, pl.ds(0, LANES)] = zv
        stripe = ACC_R // 16

        def zbody(j, carry):
            pltpu.sync_copy(zbuf, acc.at[pl.ds(s * stripe + j * LANES, LANES)])
            return carry

        lax.fori_loop(0, stripe // LANES, zbody, 0)
        plsc.subcore_barrier()

        def body(j, carry):
            pltpu.sync_copy(ones_v, acc.at[dst_v.at[j]], add=True)
            return carry

        lax.fori_loop(0, CHC, body, 0)
        plsc.subcore_barrier()
        rpt = NP // 16
        pltpu.sync_copy(acc.at[pl.ds(s * rpt, rpt)],
                        out_hbm.at[c].at[pl.ds(s * rpt, rpt)])

    return sc_count


def _build_sc_main(NP, ACC_R, CH):
    @functools.partial(
        pl.kernel,
        mesh=_MESH,
        out_type=jax.ShapeDtypeStruct((2, NP, 128), F32),
        scratch_types=[
            pltpu.VMEM_SHARED((ACC_R, 128), F32),
            pltpu.VMEM((CH, K), I32),
            pltpu.VMEM((CH, K), I32),
            pltpu.VMEM((K, 128), F32),
            pltpu.VMEM((LANES, 128), F32),
        ],
    )
    def sc_main(h_hbm, src_hbm, dst_hbm, out_hbm, acc, src_v, dst_v, rows_v, zbuf):
        c = lax.axis_index("c")
        s = lax.axis_index("s")
        pltpu.sync_copy(src_hbm.at[s], src_v)
        pltpu.sync_copy(dst_hbm.at[s], dst_v)
        zv = jnp.zeros((LANES,), F32)
        for r in range(LANES):
            for q in range(8):
                zbuf[r, pl.ds(q * LANES, LANES)] = zv
        stripe = ACC_R // 16

        def zbody(j, carry):
            pltpu.sync_copy(zbuf, acc.at[pl.ds(s * stripe + j * LANES, LANES)])
            return carry

        lax.fori_loop(0, stripe // LANES, zbody, 0)
        plsc.subcore_barrier()

        def body(j, carry):
            pltpu.sync_copy(h_hbm.at[c].at[src_v.at[j]], rows_v)
            pltpu.sync_copy(rows_v, acc.at[dst_v.at[j]], add=True)
            return carry

        lax.fori_loop(0, CH, body, 0)
        plsc.subcore_barrier()
        rpt = NP // 16

        def obody(j, carry):
            pltpu.sync_copy(acc.at[pl.ds(s * rpt + j * K, K)],
                            out_hbm.at[c].at[pl.ds(s * rpt + j * K, K)])
            return carry

        lax.fori_loop(0, rpt // K, obody, 0)

    return sc_main


def _silu(v):
    return v * (1.0 / (1.0 + jnp.exp(-v)))


def _ln(xb, g, b):
    m = jnp.mean(xb, axis=1, keepdims=True)
    d = xb - m
    v = jnp.mean(d * d, axis=1, keepdims=True)
    return d * lax.rsqrt(v + 1e-5) * g + b


def _dinv_of(cnt):
    deg = cnt[0, :, 0:1] + cnt[1, :, 0:1] + 1.0
    return lax.rsqrt(deg)


def _tc1_body(x_ref, cnt_ref, g_ref, be_ref, w_ref, hs_ref):
    dinv = _dinv_of(cnt_ref[...])
    z = _ln(x_ref[...], g_ref[...], be_ref[...])
    h = lax.dot_general(z, w_ref[...], (((1,), (1,)), ((), ())),
                        preferred_element_type=F32)
    hp = h * dinv
    hs_ref[0] = hp[:, :128]
    hs_ref[1] = hp[:, 128:]


def _tc2_body(agg_ref, hs_ref, cnt_ref, te_ref, wt_ref, bt_ref, b_ref,
              g_ref, be_ref, w_ref, hs2_ref):
    dinv = _dinv_of(cnt_ref[...])
    aggf = jnp.concatenate([agg_ref[0], agg_ref[1]], axis=1)
    hpf = jnp.concatenate([hs_ref[0], hs_ref[1]], axis=1)
    conv = dinv * (aggf + hpf) + b_ref[...]
    tvec = lax.dot_general(te_ref[...], wt_ref[...], (((1,), (1,)), ((), ())),
                           preferred_element_type=F32) + bt_ref[...]
    a = _silu(conv + tvec)
    z2 = _ln(a, g_ref[...], be_ref[...])
    h2 = lax.dot_general(z2, w_ref[...], (((1,), (1,)), ((), ())),
                         preferred_element_type=F32)
    hp2 = h2 * dinv
    hs2_ref[0] = hp2[:, :128]
    hs2_ref[1] = hp2[:, 128:]


def _tc3_body(agg_ref, hs_ref, cnt_ref, te_ref, wt_ref, bt_ref, b_ref,
              x_ref, out_ref):
    dinv = _dinv_of(cnt_ref[...])
    aggf = jnp.concatenate([agg_ref[0], agg_ref[1]], axis=1)
    hpf = jnp.concatenate([hs_ref[0], hs_ref[1]], axis=1)
    conv = dinv * (aggf + hpf) + b_ref[...]
    tvec = lax.dot_general(te_ref[...], wt_ref[...], (((1,), (1,)), ((), ())),
                           preferred_element_type=F32) + bt_ref[...]
    out_ref[...] = _silu(conv + tvec) + x_ref[...]


def _spec_rows(bn, d):
    return pl.BlockSpec((bn, d), lambda i: (i, 0))


def _spec_full(shape):
    nd = len(shape)
    return pl.BlockSpec(shape, lambda i, _nd=nd: (0,) * _nd)


def _spec_split(bn, d):
    return pl.BlockSpec((2, bn, d), lambda i: (0, i, 0))


def kernel(x, t_emb, edge_index, W1, b1, Wt1, bt1, g1, be1,
           W2, b2, Wt2, bt2, g2, be2):
    N, D = x.shape
    E = edge_index.shape[1]
    T = t_emb.shape[0]
    BN = 256
    NP = ((N + BN - 1) // BN) * BN
    TRASH = NP
    ACC_R = ((NP + 8 + 255) // 256) * 256
    GRID = NP // BN

    ei = edge_index.astype(I32)
    src, dst = ei[0], ei[1]

    ept = E // 16
    ch = (ept + K - 1) // K
    padm = ch * K - ept
    srcm = jnp.concatenate(
        [src.reshape(16, ept), jnp.zeros((16, padm), I32)], axis=1
    ).reshape(16, ch, K)
    dstm = jnp.concatenate(
        [dst.reshape(16, ept), jnp.full((16, padm), TRASH, I32)], axis=1
    ).reshape(16, ch, K)

    epc = E // 32
    chc = (epc + K - 1) // K
    padc = chc * K - epc
    dstc = jnp.concatenate(
        [dst.reshape(32, epc), jnp.full((32, padc), TRASH, I32)], axis=1
    ).reshape(32, chc, K)

    xp = jnp.pad(x, ((0, NP - N), (0, 0)))
    te = t_emb.reshape(1, T)
    b1r, bt1r, g1r, be1r = (v.reshape(1, D) for v in (b1, bt1, g1, be1))
    b2r, bt2r, g2r, be2r = (v.reshape(1, D) for v in (b2, bt2, g2, be2))

    sc_count = _build_sc_count(NP, ACC_R, chc)
    sc_main = _build_sc_main(NP, ACC_R, ch)

    cnt = sc_count(dstc)

    tc1 = pl.pallas_call(
        _tc1_body,
        grid=(GRID,),
        in_specs=[_spec_rows(BN, D), _spec_split(BN, LANES),
                  _spec_full((1, D)), _spec_full((1, D)),
                  _spec_full((D, D))],
        out_specs=_spec_split(BN, 128),
        out_shape=jax.ShapeDtypeStruct((2, NP, 128), F32),
    )
    hs1 = tc1(xp, cnt, g1r, be1r, W1)

    agg1 = sc_main(hs1, srcm, dstm)

    tc2 = pl.pallas_call(
        _tc2_body,
        grid=(GRID,),
        in_specs=[_spec_split(BN, 128), _spec_split(BN, 128),
                  _spec_split(BN, LANES), _spec_full((1, T)),
                  _spec_full((D, T)), _spec_full((1, D)), _spec_full((1, D)),
                  _spec_full((1, D)), _spec_full((1, D)), _spec_full((D, D))],
        out_specs=_spec_split(BN, 128),
        out_shape=jax.ShapeDtypeStruct((2, NP, 128), F32),
    )
    hs2 = tc2(agg1, hs1, cnt, te, Wt1, bt1r, b1r, g2r, be2r, W2)

    agg2 = sc_main(hs2, srcm, dstm)

    tc3 = pl.pallas_call(
        _tc3_body,
        grid=(GRID,),
        in_specs=[_spec_split(BN, 128), _spec_split(BN, 128),
                  _spec_split(BN, LANES), _spec_full((1, T)),
                  _spec_full((D, T)), _spec_full((1, D)), _spec_full((1, D)),
                  _spec_rows(BN, D)],
        out_specs=_spec_rows(BN, D),
        out_shape=jax.ShapeDtypeStruct((NP, D), F32),
    )
    out = tc3(agg2, hs2, cnt, te, Wt2, bt2r, b2r, xp)
    return out[:N]


# speedup vs baseline: 9.2902x; 9.2902x over previous
import functools

import jax
import jax.numpy as jnp
from jax import lax
from jax.experimental import pallas as pl
from jax.experimental.pallas import tpu as pltpu
from jax.experimental.pallas import tpu_sc as plsc

F32 = jnp.float32
I32 = jnp.int32

K = 128
LANES = 16

_MESH = plsc.VectorSubcoreMesh(core_axis_name="c", subcore_axis_name="s")


def _build_sc_count(NP, ACC_R, CHC):
    @functools.partial(
        pl.kernel,
        mesh=_MESH,
        out_type=jax.ShapeDtypeStruct((2, NP, 128), F32),
        scratch_types=[
            pltpu.VMEM_SHARED((ACC_R, 128), F32),
            pltpu.VMEM((CHC, K), I32),
            pltpu.VMEM((K, 128), F32),
            pltpu.VMEM((LANES, 128), F32),
        ],
    )
    def sc_count(dst_hbm, out_hbm, acc, dst_v, ones_v, zbuf):
        c = lax.axis_index("c")
        s = lax.axis_index("s")
        w = c * 16 + s
        pltpu.sync_copy(dst_hbm.at[w], dst_v)
        ov = jnp.ones((LANES,), F32)
        zv = jnp.zeros((LANES,), F32)
        for r in range(K):
            for q in range(8):
                ones_v[r, pl.ds(q * LANES, LANES)] = ov
        for r in range(LANES):
            for q in range(8):
                zbuf[r, pl.ds(q * LANES, LANES)] = zv
        stripe = ACC_R // 16

        def zbody(j, carry):
            pltpu.sync_copy(zbuf, acc.at[pl.ds(s * stripe + j * LANES, LANES)])
            return carry

        lax.fori_loop(0, stripe // LANES, zbody, 0)
        plsc.subcore_barrier()

        def body(j, carry):
            pltpu.sync_copy(ones_v, acc.at[dst_v.at[j]], add=True)
            return carry

        lax.fori_loop(0, CHC, body, 0)
        plsc.subcore_barrier()
        rpt = NP // 16

        def obody(j, carry):
            pltpu.sync_copy(acc.at[pl.ds(s * rpt + j * K, K)], ones_v)
            pltpu.sync_copy(ones_v, out_hbm.at[c].at[pl.ds(s * rpt + j * K, K)])
            return carry

        lax.fori_loop(0, rpt // K, obody, 0)

    return sc_count


def _build_sc_main(NP, ACC_R, CH):
    @functools.partial(
        pl.kernel,
        mesh=_MESH,
        out_type=jax.ShapeDtypeStruct((2, NP, 128), F32),
        scratch_types=[
            pltpu.VMEM_SHARED((ACC_R, 128), F32),
            pltpu.VMEM((CH, K), I32),
            pltpu.VMEM((CH, K), I32),
            pltpu.VMEM((K, 128), F32),
            pltpu.VMEM((LANES, 128), F32),
        ],
    )
    def sc_main(h_hbm, src_hbm, dst_hbm, out_hbm, acc, src_v, dst_v, rows_v, zbuf):
        c = lax.axis_index("c")
        s = lax.axis_index("s")
        pltpu.sync_copy(src_hbm.at[s], src_v)
        pltpu.sync_copy(dst_hbm.at[s], dst_v)
        zv = jnp.zeros((LANES,), F32)
        for r in range(LANES):
            for q in range(8):
                zbuf[r, pl.ds(q * LANES, LANES)] = zv
        stripe = ACC_R // 16

        def zbody(j, carry):
            pltpu.sync_copy(zbuf, acc.at[pl.ds(s * stripe + j * LANES, LANES)])
            return carry

        lax.fori_loop(0, stripe // LANES, zbody, 0)
        plsc.subcore_barrier()

        def body(j, carry):
            pltpu.sync_copy(h_hbm.at[c].at[src_v.at[j]], rows_v)
            pltpu.sync_copy(rows_v, acc.at[dst_v.at[j]], add=True)
            return carry

        lax.fori_loop(0, CH, body, 0)
        plsc.subcore_barrier()
        rpt = NP // 16

        def obody(j, carry):
            pltpu.sync_copy(acc.at[pl.ds(s * rpt + j * K, K)], rows_v)
            pltpu.sync_copy(rows_v, out_hbm.at[c].at[pl.ds(s * rpt + j * K, K)])
            return carry

        lax.fori_loop(0, rpt // K, obody, 0)

    return sc_main


def _silu(v):
    return v * (1.0 / (1.0 + jnp.exp(-v)))


def _ln(xb, g, b):
    m = jnp.mean(xb, axis=1, keepdims=True)
    d = xb - m
    v = jnp.mean(d * d, axis=1, keepdims=True)
    return d * lax.rsqrt(v + 1e-5) * g + b


def _dinv_of(cnt):
    deg = cnt[0, :, 0:1] + cnt[1, :, 0:1] + 1.0
    return lax.rsqrt(deg)


def _tc1_body(x_ref, cnt_ref, g_ref, be_ref, w_ref, hs_ref):
    dinv = _dinv_of(cnt_ref[...])
    z = _ln(x_ref[...], g_ref[...], be_ref[...])
    h = lax.dot_general(z, w_ref[...], (((1,), (1,)), ((), ())),
                        preferred_element_type=F32)
    hp = h * dinv
    hs_ref[0] = hp[:, :128]
    hs_ref[1] = hp[:, 128:]


def _tc2_body(agg_ref, hs_ref, cnt_ref, te_ref, wt_ref, bt_ref, b_ref,
              g_ref, be_ref, w_ref, hs2_ref):
    dinv = _dinv_of(cnt_ref[...])
    aggf = jnp.concatenate([agg_ref[0], agg_ref[1]], axis=1)
    hpf = jnp.concatenate([hs_ref[0], hs_ref[1]], axis=1)
    conv = dinv * (aggf + hpf) + b_ref[...]
    tvec = lax.dot_general(te_ref[...], wt_ref[...], (((1,), (1,)), ((), ())),
                           preferred_element_type=F32) + bt_ref[...]
    a = _silu(conv + tvec)
    z2 = _ln(a, g_ref[...], be_ref[...])
    h2 = lax.dot_general(z2, w_ref[...], (((1,), (1,)), ((), ())),
                         preferred_element_type=F32)
    hp2 = h2 * dinv
    hs2_ref[0] = hp2[:, :128]
    hs2_ref[1] = hp2[:, 128:]


def _tc3_body(agg_ref, hs_ref, cnt_ref, te_ref, wt_ref, bt_ref, b_ref,
              x_ref, out_ref):
    dinv = _dinv_of(cnt_ref[...])
    aggf = jnp.concatenate([agg_ref[0], agg_ref[1]], axis=1)
    hpf = jnp.concatenate([hs_ref[0], hs_ref[1]], axis=1)
    conv = dinv * (aggf + hpf) + b_ref[...]
    tvec = lax.dot_general(te_ref[...], wt_ref[...], (((1,), (1,)), ((), ())),
                           preferred_element_type=F32) + bt_ref[...]
    out_ref[...] = _silu(conv + tvec) + x_ref[...]


def _spec_rows(bn, d):
    return pl.BlockSpec((bn, d), lambda i: (i, 0))


def _spec_full(shape):
    nd = len(shape)
    return pl.BlockSpec(shape, lambda i, _nd=nd: (0,) * _nd)


def _spec_split(bn, d):
    return pl.BlockSpec((2, bn, d), lambda i: (0, i, 0))


def kernel(x, t_emb, edge_index, W1, b1, Wt1, bt1, g1, be1,
           W2, b2, Wt2, bt2, g2, be2):
    N, D = x.shape
    E = edge_index.shape[1]
    T = t_emb.shape[0]
    BN = 256
    NP = ((N + BN - 1) // BN) * BN
    TRASH = NP
    ACC_R = ((NP + 8 + 255) // 256) * 256
    GRID = NP // BN

    ei = edge_index.astype(I32)
    src, dst = ei[0], ei[1]

    ept = E // 16
    ch = (ept + K - 1) // K
    padm = ch * K - ept
    srcm = jnp.concatenate(
        [src.reshape(16, ept), jnp.zeros((16, padm), I32)], axis=1
    ).reshape(16, ch, K)
    dstm = jnp.concatenate(
        [dst.reshape(16, ept), jnp.full((16, padm), TRASH, I32)], axis=1
    ).reshape(16, ch, K)

    epc = E // 32
    chc = (epc + K - 1) // K
    padc = chc * K - epc
    dstc = jnp.concatenate(
        [dst.reshape(32, epc), jnp.full((32, padc), TRASH, I32)], axis=1
    ).reshape(32, chc, K)

    xp = jnp.pad(x, ((0, NP - N), (0, 0)))
    te = t_emb.reshape(1, T)
    b1r, bt1r, g1r, be1r = (v.reshape(1, D) for v in (b1, bt1, g1, be1))
    b2r, bt2r, g2r, be2r = (v.reshape(1, D) for v in (b2, bt2, g2, be2))

    sc_count = _build_sc_count(NP, ACC_R, chc)
    sc_main = _build_sc_main(NP, ACC_R, ch)

    cnt = sc_count(dstc)

    tc1 = pl.pallas_call(
        _tc1_body,
        grid=(GRID,),
        in_specs=[_spec_rows(BN, D), _spec_split(BN, 128),
                  _spec_full((1, D)), _spec_full((1, D)),
                  _spec_full((D, D))],
        out_specs=_spec_split(BN, 128),
        out_shape=jax.ShapeDtypeStruct((2, NP, 128), F32),
    )
    hs1 = tc1(xp, cnt, g1r, be1r, W1)

    agg1 = sc_main(hs1, srcm, dstm)

    tc2 = pl.pallas_call(
        _tc2_body,
        grid=(GRID,),
        in_specs=[_spec_split(BN, 128), _spec_split(BN, 128),
                  _spec_split(BN, 128), _spec_full((1, T)),
                  _spec_full((D, T)), _spec_full((1, D)), _spec_full((1, D)),
                  _spec_full((1, D)), _spec_full((1, D)), _spec_full((D, D))],
        out_specs=_spec_split(BN, 128),
        out_shape=jax.ShapeDtypeStruct((2, NP, 128), F32),
    )
    hs2 = tc2(agg1, hs1, cnt, te, Wt1, bt1r, b1r, g2r, be2r, W2)

    agg2 = sc_main(hs2, srcm, dstm)

    tc3 = pl.pallas_call(
        _tc3_body,
        grid=(GRID,),
        in_specs=[_spec_split(BN, 128), _spec_split(BN, 128),
                  _spec_split(BN, 128), _spec_full((1, T)),
                  _spec_full((D, T)), _spec_full((1, D)), _spec_full((1, D)),
                  _spec_rows(BN, D)],
        out_specs=_spec_rows(BN, D),
        out_shape=jax.ShapeDtypeStruct((NP, D), F32),
    )
    out = tc3(agg2, hs2, cnt, te, Wt2, bt2r, b2r, xp)
    return out[:N]


# speedup vs baseline: 10.6110x; 1.1422x over previous
import functools

import jax
import jax.numpy as jnp
from jax import lax
from jax.experimental import pallas as pl
from jax.experimental.pallas import tpu as pltpu
from jax.experimental.pallas import tpu_sc as plsc

F32 = jnp.float32
I32 = jnp.int32

K = 128
PB = 32
LANES = 16

_MESH = plsc.VectorSubcoreMesh(core_axis_name="c", subcore_axis_name="s")


def _build_sc_count(NP, ACC_R, CHC):
    @functools.partial(
        pl.kernel,
        mesh=_MESH,
        out_type=jax.ShapeDtypeStruct((2, NP, 128), F32),
        scratch_types=[
            pltpu.VMEM_SHARED((ACC_R, 128), F32),
            pltpu.VMEM((CHC, K), I32),
            pltpu.VMEM((K, 128), F32),
            pltpu.VMEM((LANES, 128), F32),
        ],
    )
    def sc_count(dst_hbm, out_hbm, acc, dst_v, ones_v, zbuf):
        c = lax.axis_index("c")
        s = lax.axis_index("s")
        w = c * 16 + s
        pltpu.sync_copy(dst_hbm.at[w], dst_v)
        ov = jnp.ones((LANES,), F32)
        zv = jnp.zeros((LANES,), F32)
        for r in range(K):
            for q in range(8):
                ones_v[r, pl.ds(q * LANES, LANES)] = ov
        for r in range(LANES):
            for q in range(8):
                zbuf[r, pl.ds(q * LANES, LANES)] = zv
        stripe = ACC_R // 16

        def zbody(j, carry):
            pltpu.sync_copy(zbuf, acc.at[pl.ds(s * stripe + j * LANES, LANES)])
            return carry

        lax.fori_loop(0, stripe // LANES, zbody, 0)
        plsc.subcore_barrier()

        def body(j, carry):
            pltpu.sync_copy(ones_v, acc.at[dst_v.at[j]], add=True)
            return carry

        lax.fori_loop(0, CHC, body, 0)
        plsc.subcore_barrier()
        rpt = NP // 16

        def obody(j, carry):
            pltpu.sync_copy(acc.at[pl.ds(s * rpt + j * K, K)], ones_v)
            pltpu.sync_copy(ones_v, out_hbm.at[c].at[pl.ds(s * rpt + j * K, K)])
            return carry

        lax.fori_loop(0, rpt // K, obody, 0)

    return sc_count


def _build_sc_main(NP, ACC_R, CH):
    @functools.partial(
        pl.kernel,
        mesh=_MESH,
        out_type=jax.ShapeDtypeStruct((2, NP, 128), F32),
        scratch_types=[
            pltpu.VMEM_SHARED((ACC_R, 128), F32),
            pltpu.VMEM((PB, K), I32),
            pltpu.VMEM((PB, K), I32),
            pltpu.VMEM((K, 128), F32),
            pltpu.VMEM((K, 128), F32),
            pltpu.SemaphoreType.DMA,
            pltpu.SemaphoreType.DMA,
        ],
    )
    def sc_main(h_hbm, src_hbm, dst_hbm, out_hbm, acc, src_v, dst_v,
                rows0, rows1, sem0, sem1):
        c = lax.axis_index("c")
        s = lax.axis_index("s")
        zv = jnp.zeros((LANES,), F32)
        for r in range(K):
            for q in range(8):
                rows0[r, pl.ds(q * LANES, LANES)] = zv
        stripe = ACC_R // 16

        def zbody(j, carry):
            pltpu.sync_copy(rows0, acc.at[pl.ds(s * stripe + j * K, K)])
            return carry

        lax.fori_loop(0, stripe // K, zbody, 0)
        if stripe % K:
            pltpu.sync_copy(rows0.at[pl.ds(0, stripe % K)],
                            acc.at[pl.ds(s * stripe + (stripe // K) * K,
                                         stripe % K)])
        plsc.subcore_barrier()

        def g_start(i, buf, sem):
            pltpu.make_async_copy(h_hbm.at[c].at[src_v.at[i]], buf, sem).start()

        def g_wait(i, buf, sem):
            pltpu.make_async_copy(h_hbm.at[c].at[src_v.at[i]], buf, sem).wait()

        for p in range((CH + PB - 1) // PB):
            base = p * PB
            nb = min(PB, CH - base)
            pltpu.sync_copy(src_hbm.at[s].at[pl.ds(base, nb)],
                            src_v.at[pl.ds(0, nb)])
            pltpu.sync_copy(dst_hbm.at[s].at[pl.ds(base, nb)],
                            dst_v.at[pl.ds(0, nb)])
            g_start(0, rows0, sem0)

            def body(i, carry, nb=nb):
                nxt = i + 1

                @pl.when(i % 2 == 0)
                def _():
                    g_wait(i, rows0, sem0)

                    @pl.when(nxt < nb)
                    def _():
                        g_start(nxt, rows1, sem1)

                    pltpu.sync_copy(rows0, acc.at[dst_v.at[i]], add=True)

                @pl.when(i % 2 == 1)
                def _():
                    g_wait(i, rows1, sem1)

                    @pl.when(nxt < nb)
                    def _():
                        g_start(nxt, rows0, sem0)

                    pltpu.sync_copy(rows1, acc.at[dst_v.at[i]], add=True)

                return carry

            lax.fori_loop(0, nb, body, 0)

        plsc.subcore_barrier()
        rpt = NP // 16

        def obody(j, carry):
            pltpu.sync_copy(acc.at[pl.ds(s * rpt + j * K, K)], rows0)
            pltpu.sync_copy(rows0, out_hbm.at[c].at[pl.ds(s * rpt + j * K, K)])
            return carry

        lax.fori_loop(0, rpt // K, obody, 0)

    return sc_main


def _silu(v):
    return v * (1.0 / (1.0 + jnp.exp(-v)))


def _ln(xb, g, b):
    m = jnp.mean(xb, axis=1, keepdims=True)
    d = xb - m
    v = jnp.mean(d * d, axis=1, keepdims=True)
    return d * lax.rsqrt(v + 1e-5) * g + b


def _dinv_of(cnt):
    deg = cnt[0, :, 0:1] + cnt[1, :, 0:1] + 1.0
    return lax.rsqrt(deg)


def _tc1_body(x_ref, cnt_ref, g_ref, be_ref, w_ref, hs_ref):
    dinv = _dinv_of(cnt_ref[...])
    z = _ln(x_ref[...], g_ref[...], be_ref[...])
    h = lax.dot_general(z, w_ref[...], (((1,), (1,)), ((), ())),
                        preferred_element_type=F32)
    hp = h * dinv
    hs_ref[0] = hp[:, :128]
    hs_ref[1] = hp[:, 128:]


def _tc2_body(agg_ref, hs_ref, cnt_ref, te_ref, wt_ref, bt_ref, b_ref,
              g_ref, be_ref, w_ref, hs2_ref):
    dinv = _dinv_of(cnt_ref[...])
    aggf = jnp.concatenate([agg_ref[0], agg_ref[1]], axis=1)
    hpf = jnp.concatenate([hs_ref[0], hs_ref[1]], axis=1)
    conv = dinv * (aggf + hpf) + b_ref[...]
    tvec = lax.dot_general(te_ref[...], wt_ref[...], (((1,), (1,)), ((), ())),
                           preferred_element_type=F32) + bt_ref[...]
    a = _silu(conv + tvec)
    z2 = _ln(a, g_ref[...], be_ref[...])
    h2 = lax.dot_general(z2, w_ref[...], (((1,), (1,)), ((), ())),
                         preferred_element_type=F32)
    hp2 = h2 * dinv
    hs2_ref[0] = hp2[:, :128]
    hs2_ref[1] = hp2[:, 128:]


def _tc3_body(agg_ref, hs_ref, cnt_ref, te_ref, wt_ref, bt_ref, b_ref,
              x_ref, out_ref):
    dinv = _dinv_of(cnt_ref[...])
    aggf = jnp.concatenate([agg_ref[0], agg_ref[1]], axis=1)
    hpf = jnp.concatenate([hs_ref[0], hs_ref[1]], axis=1)
    conv = dinv * (aggf + hpf) + b_ref[...]
    tvec = lax.dot_general(te_ref[...], wt_ref[...], (((1,), (1,)), ((), ())),
                           preferred_element_type=F32) + bt_ref[...]
    out_ref[...] = _silu(conv + tvec) + x_ref[...]


def _spec_rows(bn, d):
    return pl.BlockSpec((bn, d), lambda i: (i, 0))


def _spec_full(shape):
    nd = len(shape)
    return pl.BlockSpec(shape, lambda i, _nd=nd: (0,) * _nd)


def _spec_split(bn, d):
    return pl.BlockSpec((2, bn, d), lambda i: (0, i, 0))


def kernel(x, t_emb, edge_index, W1, b1, Wt1, bt1, g1, be1,
           W2, b2, Wt2, bt2, g2, be2):
    N, D = x.shape
    E = edge_index.shape[1]
    T = t_emb.shape[0]
    BN = 256
    NP = ((N + BN - 1) // BN) * BN
    TRASH = NP
    ACC_R = ((NP + 16 + 15) // 16) * 16
    GRID = NP // BN

    ei = edge_index.astype(I32)
    src, dst = ei[0], ei[1]

    ept = E // 16
    ch = (ept + K - 1) // K
    padm = ch * K - ept
    srcm = jnp.concatenate(
        [src.reshape(16, ept), jnp.zeros((16, padm), I32)], axis=1
    ).reshape(16, ch, K)
    dstm = jnp.concatenate(
        [dst.reshape(16, ept), jnp.full((16, padm), TRASH, I32)], axis=1
    ).reshape(16, ch, K)

    epc = E // 32
    chc = (epc + K - 1) // K
    padc = chc * K - epc
    dstc = jnp.concatenate(
        [dst.reshape(32, epc), jnp.full((32, padc), TRASH, I32)], axis=1
    ).reshape(32, chc, K)

    xp = jnp.pad(x, ((0, NP - N), (0, 0)))
    te = t_emb.reshape(1, T)
    b1r, bt1r, g1r, be1r = (v.reshape(1, D) for v in (b1, bt1, g1, be1))
    b2r, bt2r, g2r, be2r = (v.reshape(1, D) for v in (b2, bt2, g2, be2))

    sc_count = _build_sc_count(NP, ACC_R, chc)
    sc_main = _build_sc_main(NP, ACC_R, ch)

    cnt = sc_count(dstc)

    tc1 = pl.pallas_call(
        _tc1_body,
        grid=(GRID,),
        in_specs=[_spec_rows(BN, D), _spec_split(BN, 128),
                  _spec_full((1, D)), _spec_full((1, D)),
                  _spec_full((D, D))],
        out_specs=_spec_split(BN, 128),
        out_shape=jax.ShapeDtypeStruct((2, NP, 128), F32),
    )
    hs1 = tc1(xp, cnt, g1r, be1r, W1)

    agg1 = sc_main(hs1, srcm, dstm)

    tc2 = pl.pallas_call(
        _tc2_body,
        grid=(GRID,),
        in_specs=[_spec_split(BN, 128), _spec_split(BN, 128),
                  _spec_split(BN, 128), _spec_full((1, T)),
                  _spec_full((D, T)), _spec_full((1, D)), _spec_full((1, D)),
                  _spec_full((1, D)), _spec_full((1, D)), _spec_full((D, D))],
        out_specs=_spec_split(BN, 128),
        out_shape=jax.ShapeDtypeStruct((2, NP, 128), F32),
    )
    hs2 = tc2(agg1, hs1, cnt, te, Wt1, bt1r, b1r, g2r, be2r, W2)

    agg2 = sc_main(hs2, srcm, dstm)

    tc3 = pl.pallas_call(
        _tc3_body,
        grid=(GRID,),
        in_specs=[_spec_split(BN, 128), _spec_split(BN, 128),
                  _spec_split(BN, 128), _spec_full((1, T)),
                  _spec_full((D, T)), _spec_full((1, D)), _spec_full((1, D)),
                  _spec_rows(BN, D)],
        out_specs=_spec_rows(BN, D),
        out_shape=jax.ShapeDtypeStruct((NP, D), F32),
    )
    out = tc3(agg2, hs2, cnt, te, Wt2, bt2r, b2r, xp)
    return out[:N]


# speedup vs baseline: 10.7516x; 1.0133x over previous
import functools

import jax
import jax.numpy as jnp
from jax import lax
from jax.experimental import pallas as pl
from jax.experimental.pallas import tpu as pltpu
from jax.experimental.pallas import tpu_sc as plsc

F32 = jnp.float32
I32 = jnp.int32

K = 128
PB = 32
LANES = 16

_MESH = plsc.VectorSubcoreMesh(core_axis_name="c", subcore_axis_name="s")


def _build_sc_count(NP, ACC_R, CHC):
    @functools.partial(
        pl.kernel,
        mesh=_MESH,
        out_type=jax.ShapeDtypeStruct((2, NP, 128), F32),
        scratch_types=[
            pltpu.VMEM_SHARED((ACC_R, 128), F32),
            pltpu.VMEM((CHC, K), I32),
            pltpu.VMEM((K, 128), F32),
            pltpu.VMEM((LANES, 128), F32),
        ],
    )
    def sc_count(dst_hbm, out_hbm, acc, dst_v, ones_v, zbuf):
        c = lax.axis_index("c")
        s = lax.axis_index("s")
        w = c * 16 + s
        pltpu.sync_copy(dst_hbm.at[w], dst_v)
        ov = jnp.ones((LANES,), F32)
        zv = jnp.zeros((LANES,), F32)
        for r in range(K):
            for q in range(8):
                ones_v[r, pl.ds(q * LANES, LANES)] = ov
        for r in range(LANES):
            for q in range(8):
                zbuf[r, pl.ds(q * LANES, LANES)] = zv
        stripe = ACC_R // 16

        def zbody(j, carry):
            pltpu.sync_copy(zbuf, acc.at[pl.ds(s * stripe + j * LANES, LANES)])
            return carry

        lax.fori_loop(0, stripe // LANES, zbody, 0)
        plsc.subcore_barrier()

        def body(j, carry):
            pltpu.sync_copy(ones_v, acc.at[dst_v.at[j]], add=True)
            return carry

        lax.fori_loop(0, CHC, body, 0)
        plsc.subcore_barrier()
        rpt = NP // 16

        def obody(j, carry):
            pltpu.sync_copy(acc.at[pl.ds(s * rpt + j * K, K)], ones_v)
            pltpu.sync_copy(ones_v, out_hbm.at[c].at[pl.ds(s * rpt + j * K, K)])
            return carry

        lax.fori_loop(0, rpt // K, obody, 0)

    return sc_count


def _build_sc_main(NP, ACC_R, CH):
    @functools.partial(
        pl.kernel,
        mesh=_MESH,
        out_type=jax.ShapeDtypeStruct((2, NP, 128), F32),
        scratch_types=[
            pltpu.VMEM_SHARED((ACC_R, 128), F32),
            pltpu.VMEM((PB, K), I32),
            pltpu.VMEM((PB, K), I32),
            pltpu.VMEM((K, 128), F32),
            pltpu.VMEM((K, 128), F32),
            pltpu.SemaphoreType.DMA,
            pltpu.SemaphoreType.DMA,
        ],
    )
    def sc_main(h_hbm, src_hbm, dst_hbm, out_hbm, acc, src_v, dst_v,
                rows0, rows1, sem0, sem1):
        c = lax.axis_index("c")
        s = lax.axis_index("s")
        zv = jnp.zeros((LANES,), F32)
        for r in range(K):
            for q in range(8):
                rows0[r, pl.ds(q * LANES, LANES)] = zv
        stripe = ACC_R // 16

        def zbody(j, carry):
            pltpu.sync_copy(rows0, acc.at[pl.ds(s * stripe + j * K, K)])
            return carry

        lax.fori_loop(0, stripe // K, zbody, 0)
        if stripe % K:
            pltpu.sync_copy(rows0.at[pl.ds(0, stripe % K)],
                            acc.at[pl.ds(s * stripe + (stripe // K) * K,
                                         stripe % K)])
        plsc.subcore_barrier()

        def g_start(i, buf, sem):
            pltpu.make_async_copy(h_hbm.at[c].at[src_v.at[i]], buf, sem).start()

        def g_wait(i, buf, sem):
            pltpu.make_async_copy(h_hbm.at[c].at[src_v.at[i]], buf, sem).wait()

        for p in range((CH + PB - 1) // PB):
            base = p * PB
            nb = min(PB, CH - base)
            pltpu.sync_copy(src_hbm.at[s].at[pl.ds(base, nb)],
                            src_v.at[pl.ds(0, nb)])
            pltpu.sync_copy(dst_hbm.at[s].at[pl.ds(base, nb)],
                            dst_v.at[pl.ds(0, nb)])
            g_start(0, rows0, sem0)

            def body(i, carry, nb=nb):
                nxt = i + 1

                @pl.when(i % 2 == 0)
                def _():
                    g_wait(i, rows0, sem0)

                    @pl.when(nxt < nb)
                    def _():
                        g_start(nxt, rows1, sem1)

                    pltpu.sync_copy(rows0, acc.at[dst_v.at[i]], add=True)

                @pl.when(i % 2 == 1)
                def _():
                    g_wait(i, rows1, sem1)

                    @pl.when(nxt < nb)
                    def _():
                        g_start(nxt, rows0, sem0)

                    pltpu.sync_copy(rows1, acc.at[dst_v.at[i]], add=True)

                return carry

            lax.fori_loop(0, nb, body, 0)

        plsc.subcore_barrier()
        rpt = NP // 16

        def obody(j, carry):
            pltpu.sync_copy(acc.at[pl.ds(s * rpt + j * K, K)], rows0)
            pltpu.sync_copy(rows0, out_hbm.at[c].at[pl.ds(s * rpt + j * K, K)])
            return carry

        lax.fori_loop(0, rpt // K, obody, 0)

    return sc_main


def _silu(v):
    return v * (1.0 / (1.0 + jnp.exp(-v)))


def _ln(xb, g, b):
    m = jnp.mean(xb, axis=1, keepdims=True)
    d = xb - m
    v = jnp.mean(d * d, axis=1, keepdims=True)
    return d * lax.rsqrt(v + 1e-5) * g + b


def _tc1a_body(x_ref, g_ref, be_ref, w_ref, h_ref):
    z = _ln(x_ref[...], g_ref[...], be_ref[...])
    h_ref[...] = lax.dot_general(z, w_ref[...], (((1,), (1,)), ((), ())),
                                 preferred_element_type=F32)


def _tc1b_body(h_ref, cnt_ref, hs_ref, dinv_ref):
    cnt = cnt_ref[...]
    deg = cnt[0, :, 0:1] + cnt[1, :, 0:1] + 1.0
    dinv = lax.rsqrt(deg)
    dinv_ref[...] = jnp.broadcast_to(dinv, dinv_ref.shape)
    hp = h_ref[...] * dinv
    hs_ref[0] = hp[:, :128]
    hs_ref[1] = hp[:, 128:]


def _tc2_body(agg_ref, hs_ref, dinv_ref, te_ref, wt_ref, bt_ref, b_ref,
              g_ref, be_ref, w_ref, hs2_ref):
    dinv = dinv_ref[...][:, 0:1]
    aggf = jnp.concatenate([agg_ref[0], agg_ref[1]], axis=1)
    hpf = jnp.concatenate([hs_ref[0], hs_ref[1]], axis=1)
    conv = dinv * (aggf + hpf) + b_ref[...]
    tvec = lax.dot_general(te_ref[...], wt_ref[...], (((1,), (1,)), ((), ())),
                           preferred_element_type=F32) + bt_ref[...]
    a = _silu(conv + tvec)
    z2 = _ln(a, g_ref[...], be_ref[...])
    h2 = lax.dot_general(z2, w_ref[...], (((1,), (1,)), ((), ())),
                         preferred_element_type=F32)
    hp2 = h2 * dinv
    hs2_ref[0] = hp2[:, :128]
    hs2_ref[1] = hp2[:, 128:]


def _tc3_body(agg_ref, hs_ref, dinv_ref, te_ref, wt_ref, bt_ref, b_ref,
              x_ref, out_ref):
    dinv = dinv_ref[...][:, 0:1]
    aggf = jnp.concatenate([agg_ref[0], agg_ref[1]], axis=1)
    hpf = jnp.concatenate([hs_ref[0], hs_ref[1]], axis=1)
    conv = dinv * (aggf + hpf) + b_ref[...]
    tvec = lax.dot_general(te_ref[...], wt_ref[...], (((1,), (1,)), ((), ())),
                           preferred_element_type=F32) + bt_ref[...]
    out_ref[...] = _silu(conv + tvec) + x_ref[...]


def _spec_rows(bn, d):
    return pl.BlockSpec((bn, d), lambda i: (i, 0))


def _spec_full(shape):
    nd = len(shape)
    return pl.BlockSpec(shape, lambda i, _nd=nd: (0,) * _nd)


def _spec_split(bn, d):
    return pl.BlockSpec((2, bn, d), lambda i: (0, i, 0))


def kernel(x, t_emb, edge_index, W1, b1, Wt1, bt1, g1, be1,
           W2, b2, Wt2, bt2, g2, be2):
    N, D = x.shape
    E = edge_index.shape[1]
    T = t_emb.shape[0]
    BN = 256
    NP = ((N + BN - 1) // BN) * BN
    TRASH = NP
    ACC_R = ((NP + 16 + 15) // 16) * 16
    GRID = NP // BN

    ei = edge_index.astype(I32)
    src, dst = ei[0], ei[1]

    ept = E // 16
    ch = (ept + K - 1) // K
    padm = ch * K - ept
    srcm = jnp.concatenate(
        [src.reshape(16, ept), jnp.zeros((16, padm), I32)], axis=1
    ).reshape(16, ch, K)
    dstm = jnp.concatenate(
        [dst.reshape(16, ept), jnp.full((16, padm), TRASH, I32)], axis=1
    ).reshape(16, ch, K)

    epc = E // 32
    chc = (epc + K - 1) // K
    padc = chc * K - epc
    dstc = jnp.concatenate(
        [dst.reshape(32, epc), jnp.full((32, padc), TRASH, I32)], axis=1
    ).reshape(32, chc, K)

    xp = jnp.pad(x, ((0, NP - N), (0, 0)))
    te = t_emb.reshape(1, T)
    b1r, bt1r, g1r, be1r = (v.reshape(1, D) for v in (b1, bt1, g1, be1))
    b2r, bt2r, g2r, be2r = (v.reshape(1, D) for v in (b2, bt2, g2, be2))

    sc_count = _build_sc_count(NP, ACC_R, chc)
    sc_main = _build_sc_main(NP, ACC_R, ch)

    cnt = sc_count(dstc)

    tc1a = pl.pallas_call(
        _tc1a_body,
        grid=(GRID,),
        in_specs=[_spec_rows(BN, D), _spec_full((1, D)), _spec_full((1, D)),
                  _spec_full((D, D))],
        out_specs=_spec_rows(BN, D),
        out_shape=jax.ShapeDtypeStruct((NP, D), F32),
    )
    h1raw = tc1a(xp, g1r, be1r, W1)

    tc1b = pl.pallas_call(
        _tc1b_body,
        grid=(GRID,),
        in_specs=[_spec_rows(BN, D), _spec_split(BN, 128)],
        out_specs=[_spec_split(BN, 128), _spec_rows(BN, 128)],
        out_shape=[jax.ShapeDtypeStruct((2, NP, 128), F32),
                   jax.ShapeDtypeStruct((NP, 128), F32)],
    )
    hs1, dinvb = tc1b(h1raw, cnt)

    agg1 = sc_main(hs1, srcm, dstm)

    tc2 = pl.pallas_call(
        _tc2_body,
        grid=(GRID,),
        in_specs=[_spec_split(BN, 128), _spec_split(BN, 128),
                  _spec_rows(BN, 128), _spec_full((1, T)),
                  _spec_full((D, T)), _spec_full((1, D)), _spec_full((1, D)),
                  _spec_full((1, D)), _spec_full((1, D)), _spec_full((D, D))],
        out_specs=_spec_split(BN, 128),
        out_shape=jax.ShapeDtypeStruct((2, NP, 128), F32),
    )
    hs2 = tc2(agg1, hs1, dinvb, te, Wt1, bt1r, b1r, g2r, be2r, W2)

    agg2 = sc_main(hs2, srcm, dstm)

    tc3 = pl.pallas_call(
        _tc3_body,
        grid=(GRID,),
        in_specs=[_spec_split(BN, 128), _spec_split(BN, 128),
                  _spec_rows(BN, 128), _spec_full((1, T)),
                  _spec_full((D, T)), _spec_full((1, D)), _spec_full((1, D)),
                  _spec_rows(BN, D)],
        out_specs=_spec_rows(BN, D),
        out_shape=jax.ShapeDtypeStruct((NP, D), F32),
    )
    out = tc3(agg2, hs2, dinvb, te, Wt2, bt2r, b2r, xp)
    return out[:N]
